# Initial kernel scaffold; baseline (speedup 1.0000x reference)
import functools

import jax
import jax.numpy as jnp
from jax import lax
from jax.experimental import pallas as pl
from jax.experimental.pallas import tpu as pltpu
from jax.experimental.pallas import tpu_sc as plsc

N = 10000
E = 320000
D = 128

NW = 32
BATCH = 128
NB = 79
E_PAD = NW * NB * BATCH
NCHUNK = E_PAD // BATCH
ROWS_PER_TILE = N // 16


def _sc_body(row_h, col_h, ew_h, feats_h, acc_o, deg_o,
             ridx, cidx, wv, rowsb, wwide, acc_sh, deg_sh, sem):
    cid = lax.axis_index("c")
    sid = lax.axis_index("s")
    wid = cid * 16 + sid

    pltpu.sync_copy(row_h.at[pl.ds(wid * NB, NB)], ridx)
    pltpu.sync_copy(col_h.at[pl.ds(wid * NB, NB)], cidx)
    pltpu.sync_copy(ew_h.at[pl.ds(wid * NB, NB)], wv)

    zero16 = jnp.zeros((16,), jnp.float32)

    @pl.loop(0, BATCH)
    def _zero_vmem(i):
        for c8 in range(D // 16):
            rowsb[i, pl.ds(c8 * 16, 16)] = zero16
        wwide[i, :] = zero16

    @pl.loop(0, 5)
    def _zero_shared(j):
        base = sid * ROWS_PER_TILE + j * 125
        pltpu.sync_copy(rowsb.at[pl.ds(0, 125)], acc_sh.at[pl.ds(base, 125)])
        pltpu.sync_copy(wwide.at[pl.ds(0, 125)], deg_sh.at[pl.ds(base, 125)])

    plsc.subcore_barrier()

    lane_iota = lax.iota(jnp.int32, 16)
    zero_idx = jnp.zeros((16,), jnp.int32)

    @pl.loop(0, NB)
    def _edge_batch(b):
        pltpu.async_copy(feats_h.at[cidx.at[b]], rowsb, sem).wait()

        @pl.loop(0, BATCH // 16)
        def _mul_group(g):
            base = g * 16
            for e in range(16):
                ws = wv[b, base + e]
                wb = jnp.full((16,), ws, jnp.float32)
                for c8 in range(D // 16):
                    sl = pl.ds(c8 * 16, 16)
                    rowsb[base + e, sl] = rowsb[base + e, sl] * wb
            w16 = wv[b, pl.ds(base, 16)]
            plsc.store_scatter(wwide, [base + lane_iota, zero_idx], w16)

        pltpu.sync_copy(rowsb, acc_sh.at[ridx.at[b]], add=True)
        pltpu.sync_copy(wwide, deg_sh.at[ridx.at[b]], add=True)

    plsc.subcore_barrier()

    r0 = sid * ROWS_PER_TILE
    pltpu.sync_copy(acc_sh.at[pl.ds(r0, ROWS_PER_TILE)],
                    acc_o.at[cid, pl.ds(r0, ROWS_PER_TILE)])
    pltpu.sync_copy(deg_sh.at[pl.ds(r0, ROWS_PER_TILE)],
                    deg_o.at[cid, pl.ds(r0, ROWS_PER_TILE)])


_sc_agg = functools.partial(
    pl.kernel,
    out_type=(jax.ShapeDtypeStruct((2, N, D), jnp.float32),
              jax.ShapeDtypeStruct((2, N, 16), jnp.float32)),
    mesh=plsc.VectorSubcoreMesh(core_axis_name="c", subcore_axis_name="s"),
    scratch_types=[
        pltpu.VMEM((NB, BATCH), jnp.int32),
        pltpu.VMEM((NB, BATCH), jnp.int32),
        pltpu.VMEM((NB, BATCH), jnp.float32),
        pltpu.VMEM((BATCH, D), jnp.float32),
        pltpu.VMEM((BATCH, 16), jnp.float32),
        pltpu.VMEM_SHARED((N, D), jnp.float32),
        pltpu.VMEM_SHARED((N, 16), jnp.float32),
        pltpu.SemaphoreType.DMA,
    ],
)(_sc_body)


BLK = 2000


def _tc_body(a0, a1, d0, d1, f, w, bb, o):
    agg = (a0[...] + a1[...]) / (d0[...] + d1[...])
    h = lax.dot_general(agg, w[...], (((1,), (1,)), ((), ())),
                        preferred_element_type=jnp.float32)
    o[...] = f[...] + jnp.maximum(h + bb[...], 0.0)


def _tc_post(acc0, acc1, deg0, deg1, feats, W, b2):
    return pl.pallas_call(
        _tc_body,
        grid=(N // BLK,),
        in_specs=[
            pl.BlockSpec((BLK, D), lambda i: (i, 0)),
            pl.BlockSpec((BLK, D), lambda i: (i, 0)),
            pl.BlockSpec((BLK, 1), lambda i: (i, 0)),
            pl.BlockSpec((BLK, 1), lambda i: (i, 0)),
            pl.BlockSpec((BLK, D), lambda i: (i, 0)),
            pl.BlockSpec((D, D), lambda i: (0, 0)),
            pl.BlockSpec((1, D), lambda i: (0, 0)),
        ],
        out_specs=pl.BlockSpec((BLK, D), lambda i: (i, 0)),
        out_shape=jax.ShapeDtypeStruct((N, D), jnp.float32),
    )(acc0, acc1, deg0, deg1, feats, W, b2)


@jax.jit
def kernel(edge_index, edge_weight, feats, W, b):
    row = edge_index[0].astype(jnp.int32)
    col = edge_index[1].astype(jnp.int32)
    ew = edge_weight.astype(jnp.float32)
    pad = E_PAD - E
    row2 = jnp.concatenate([row, jnp.zeros((pad,), jnp.int32)]).reshape(NCHUNK, BATCH)
    col2 = jnp.concatenate([col, jnp.zeros((pad,), jnp.int32)]).reshape(NCHUNK, BATCH)
    ew2 = jnp.concatenate([ew, jnp.zeros((pad,), jnp.float32)]).reshape(NCHUNK, BATCH)

    acc, deg = _sc_agg(row2, col2, ew2, feats)
    return _tc_post(acc[0], acc[1], deg[0, :, :1], deg[1, :, :1],
                    feats, W, b.reshape(1, D))


# speedup vs baseline: 5.2335x; 5.2335x over previous
import functools

import jax
import jax.numpy as jnp
from jax import lax
from jax.experimental import pallas as pl
from jax.experimental.pallas import tpu as pltpu
from jax.experimental.pallas import tpu_sc as plsc

N = 10000
E = 320000
D = 128

NW = 32
BATCH = 128
NB = 79
E_PAD = NW * NB * BATCH
N_PAD = 10240
ROWS_PER_TILE = N_PAD // 16


def _sc_body(row_h, col_h, ew_h, feats_h, acc_o, deg_o,
             ridx, cidx, wv, rowsb, zv, acc_sh, deg_sh, sem):
    cid = lax.axis_index("c")
    sid = lax.axis_index("s")
    wid = cid * 16 + sid

    pltpu.sync_copy(row_h.at[wid], ridx)
    pltpu.sync_copy(col_h.at[wid], cidx)
    pltpu.sync_copy(ew_h.at[wid], wv)

    zero16 = jnp.zeros((16,), jnp.float32)

    @pl.loop(0, BATCH)
    def _zero_vmem(i):
        for c8 in range(D // 16):
            rowsb[i, pl.ds(c8 * 16, 16)] = zero16

    @pl.loop(0, ROWS_PER_TILE // 16)
    def _zero_zv(i):
        zv[pl.ds(i * 16, 16)] = zero16

    @pl.loop(0, ROWS_PER_TILE // BATCH)
    def _zero_shared(j):
        base = sid * ROWS_PER_TILE + j * BATCH
        pltpu.sync_copy(rowsb, acc_sh.at[pl.ds(base, BATCH)])

    pltpu.sync_copy(zv, deg_sh.at[pl.ds(sid * ROWS_PER_TILE, ROWS_PER_TILE)])

    plsc.subcore_barrier()

    @pl.loop(0, NB)
    def _edge_batch(b):
        pltpu.async_copy(feats_h.at[cidx.at[b]], rowsb, sem).wait()

        @pl.loop(0, BATCH // 16)
        def _mul_group(g):
            base = g * 16
            w16 = wv[b, pl.ds(base, 16)]
            for e in range(16):
                wb = jnp.full((16,), w16[e], jnp.float32)
                for c8 in range(D // 16):
                    sl = pl.ds(c8 * 16, 16)
                    rowsb[base + e, sl] = rowsb[base + e, sl] * wb

        pltpu.sync_copy(rowsb, acc_sh.at[ridx.at[b]], add=True)
        pltpu.sync_copy(wv.at[b], deg_sh.at[ridx.at[b]], add=True)

    plsc.subcore_barrier()

    r0 = sid * ROWS_PER_TILE
    pltpu.sync_copy(acc_sh.at[pl.ds(r0, ROWS_PER_TILE)],
                    acc_o.at[cid, pl.ds(r0, ROWS_PER_TILE)])
    pltpu.sync_copy(deg_sh.at[pl.ds(r0, ROWS_PER_TILE)],
                    deg_o.at[cid, pl.ds(r0, ROWS_PER_TILE)])


_sc_agg = functools.partial(
    pl.kernel,
    out_type=(jax.ShapeDtypeStruct((2, N_PAD, D), jnp.float32),
              jax.ShapeDtypeStruct((2, N_PAD), jnp.float32)),
    mesh=plsc.VectorSubcoreMesh(core_axis_name="c", subcore_axis_name="s"),
    scratch_types=[
        pltpu.VMEM((NB, BATCH), jnp.int32),
        pltpu.VMEM((NB, BATCH), jnp.int32),
        pltpu.VMEM((NB, BATCH), jnp.float32),
        pltpu.VMEM((BATCH, D), jnp.float32),
        pltpu.VMEM((ROWS_PER_TILE,), jnp.float32),
        pltpu.VMEM_SHARED((N_PAD, D), jnp.float32),
        pltpu.VMEM_SHARED((N_PAD,), jnp.float32),
        pltpu.SemaphoreType.DMA,
    ],
)(_sc_body)


BLK = 2000


def _tc_body(a0, a1, d0, d1, f, w, bb, o):
    agg = (a0[...] + a1[...]) / (d0[...] + d1[...])
    h = lax.dot_general(agg, w[...], (((1,), (1,)), ((), ())),
                        preferred_element_type=jnp.float32)
    o[...] = f[...] + jnp.maximum(h + bb[...], 0.0)


def _tc_post(acc0, acc1, deg0, deg1, feats, W, b2):
    return pl.pallas_call(
        _tc_body,
        grid=(N // BLK,),
        in_specs=[
            pl.BlockSpec((BLK, D), lambda i: (i, 0)),
            pl.BlockSpec((BLK, D), lambda i: (i, 0)),
            pl.BlockSpec((BLK, 1), lambda i: (i, 0)),
            pl.BlockSpec((BLK, 1), lambda i: (i, 0)),
            pl.BlockSpec((BLK, D), lambda i: (i, 0)),
            pl.BlockSpec((D, D), lambda i: (0, 0)),
            pl.BlockSpec((1, D), lambda i: (0, 0)),
        ],
        out_specs=pl.BlockSpec((BLK, D), lambda i: (i, 0)),
        out_shape=jax.ShapeDtypeStruct((N, D), jnp.float32),
    )(acc0, acc1, deg0, deg1, feats, W, b2)


@jax.jit
def kernel(edge_index, edge_weight, feats, W, b):
    row = edge_index[0].astype(jnp.int32)
    col = edge_index[1].astype(jnp.int32)
    ew = edge_weight.astype(jnp.float32)
    pad = E_PAD - E
    row2 = jnp.concatenate([row, jnp.zeros((pad,), jnp.int32)]).reshape(NW, NB, BATCH)
    col2 = jnp.concatenate([col, jnp.zeros((pad,), jnp.int32)]).reshape(NW, NB, BATCH)
    ew2 = jnp.concatenate([ew, jnp.zeros((pad,), jnp.float32)]).reshape(NW, NB, BATCH)

    acc, deg = _sc_agg(row2, col2, ew2, feats)
    return _tc_post(acc[0], acc[1], deg[0].reshape(N_PAD, 1),
                    deg[1].reshape(N_PAD, 1), feats, W, b.reshape(1, D))
